# Initial kernel scaffold; baseline (speedup 1.0000x reference)
import jax
import jax.numpy as jnp
from jax.experimental import pallas as pl


def kernel(h, x, edge_index, edge_attr, em_w1, em_b1, em_w2, em_b2, ei_w, ei_b, xm_w1, xm_b1, xm_w2, nm_w1, nm_b1, nm_w2, nm_b2):
    raise NotImplementedError("write your pallas kernel here")


# speedup vs baseline: 2.9070x; 2.9070x over previous
import functools

import jax
import jax.numpy as jnp
from jax import lax
from jax.experimental import pallas as pl
from jax.experimental.pallas import tpu as pltpu
from jax.experimental.pallas import tpu_sc as plsc

N = 10000
E = 320000
H = 128
DE = 16

NC = 2
NS = 16
NW = NC * NS
EPW = E // NW
CHUNK = 80
NCHUNK = EPW // CHUNK
NPS = N // NS
ZROWS = 125
assert NPS % ZROWS == 0


def _silu(v):
    return v * jax.nn.sigmoid(v)


def _tables_body(h_ref, w1a_ref, w1b_ref, b1_ref, tdst_ref, tsrc_ref):
    h = h_ref[...]
    tdst_ref[...] = jnp.dot(h, w1a_ref[...],
                            preferred_element_type=jnp.float32) + b1_ref[...]
    tsrc_ref[...] = jnp.dot(h, w1b_ref[...],
                            preferred_element_type=jnp.float32)


def _make_tables(h, w1a, w1b, b1):
    return pl.pallas_call(
        _tables_body,
        out_shape=(jax.ShapeDtypeStruct((N, H), jnp.float32),
                   jax.ShapeDtypeStruct((N, H), jnp.float32)),
    )(h, w1a, w1b, b1)


def _gather_body(tdh_ref, tsh_ref, x4f_ref, dsti_ref, srci_ref,
                 gdh_ref, gsh_ref, gxf_ref,
                 dstv, srcv, bufd, bufs, x4v, bx1, sem):
    wid = lax.axis_index("s") * NC + lax.axis_index("c")
    base = wid * EPW
    pltpu.sync_copy(dsti_ref.at[pl.ds(base, EPW)], dstv)
    pltpu.sync_copy(srci_ref.at[pl.ds(base, EPW)], srcv)
    pltpu.sync_copy(x4f_ref, x4v)

    lane4 = lax.iota(jnp.int32, 16) * 4

    def body(j, carry):
        off = j * CHUNK
        cd = pltpu.async_copy(tdh_ref.at[dstv.at[pl.ds(off, CHUNK)]], bufd, sem)
        cs = pltpu.async_copy(tsh_ref.at[srcv.at[pl.ds(off, CHUNK)]], bufs, sem)
        for g in range(CHUNK // 16):
            di = dstv[pl.ds(off + 16 * g, 16)]
            si = srcv[pl.ds(off + 16 * g, 16)]
            dsq = jnp.zeros((16,), jnp.float32)
            rels = []
            for c in range(3):
                xd = plsc.load_gather(x4v, [di * 4 + c])
                xs = plsc.load_gather(x4v, [si * 4 + c])
                r = xd - xs
                rels.append(r)
                dsq = dsq + r * r
            pos = lane4 + g * 64
            plsc.store_scatter(bx1, [pos], dsq)
            for c in range(3):
                plsc.store_scatter(bx1, [pos + (c + 1)], rels[c])
        cd.wait()
        cs.wait()
        pltpu.sync_copy(bufd, gdh_ref.at[pl.ds(base + off, CHUNK)])
        pltpu.sync_copy(bufs, gsh_ref.at[pl.ds(base + off, CHUNK)])
        pltpu.sync_copy(bx1, gxf_ref.at[pl.ds((base + off) * 4, CHUNK * 4)])
        return carry

    lax.fori_loop(0, NCHUNK, body, 0)


def _sc_gather(tdh, tsh, x4f, dsti, srci):
    mesh = plsc.VectorSubcoreMesh(core_axis_name="c", subcore_axis_name="s",
                                  num_cores=NC, num_subcores=NS)
    return pl.kernel(
        _gather_body,
        out_type=(jax.ShapeDtypeStruct((E, H), jnp.float32),
                  jax.ShapeDtypeStruct((E, H), jnp.float32),
                  jax.ShapeDtypeStruct((E * 4,), jnp.float32)),
        mesh=mesh,
        scratch_types=[
            pltpu.VMEM((EPW,), jnp.int32),
            pltpu.VMEM((EPW,), jnp.int32),
            pltpu.VMEM((CHUNK, H), jnp.float32),
            pltpu.VMEM((CHUNK, H), jnp.float32),
            pltpu.VMEM((N * 4,), jnp.float32),
            pltpu.VMEM((CHUNK * 4,), jnp.float32),
            pltpu.SemaphoreType.DMA,
        ],
        compiler_params=pltpu.CompilerParams(needs_layout_passes=False),
    )(tdh, tsh, x4f, dsti, srci)


def _edge_body(gdh_ref, gsh_ref, gx_ref, ea_ref, wef_ref, w2_ref, b2_ref,
               eiw_ref, eib_ref, xw1_ref, xb1_ref, xw2_ref,
               msc_ref, mdx_ref):
    pre = gdh_ref[...] + gsh_ref[...]
    gx = gx_ref[...]
    d_sq = gx[:, 0:1]
    rel = gx[:, 1:4]
    eb = pre.shape[0]
    ef = jnp.concatenate(
        [d_sq, ea_ref[...], jnp.zeros((eb, 7), jnp.float32)], axis=1)
    a1 = _silu(pre + jnp.dot(ef, wef_ref[...],
                             preferred_element_type=jnp.float32))
    mij = jnp.dot(a1, w2_ref[...], preferred_element_type=jnp.float32) \
        + b2_ref[...]
    eij = jax.nn.sigmoid(
        jnp.dot(mij, eiw_ref[...], preferred_element_type=jnp.float32)[:, :1]
        + eib_ref[0, 0])
    msc_ref[...] = mij * eij
    t = jnp.dot(_silu(jnp.dot(mij, xw1_ref[...],
                              preferred_element_type=jnp.float32)
                      + xb1_ref[...]),
                xw2_ref[...], preferred_element_type=jnp.float32)[:, :1]
    xg = jnp.tanh(t)
    scale = xg / (jnp.sqrt(d_sq + 1e-8) + 1.0)
    mdx_ref[...] = jnp.concatenate(
        [rel * scale, jnp.zeros((eb, 13), jnp.float32)], axis=1)


def _edge_mlp(gdh, gsh, gx, edge_attr, wef, w2, b2, eiw, eib, xw1, xb1, xw2):
    EB = 2000
    grid = (E // EB,)
    return pl.pallas_call(
        _edge_body,
        grid=grid,
        in_specs=[
            pl.BlockSpec((EB, H), lambda i: (i, 0)),
            pl.BlockSpec((EB, H), lambda i: (i, 0)),
            pl.BlockSpec((EB, 4), lambda i: (i, 0)),
            pl.BlockSpec((EB, DE), lambda i: (i, 0)),
            pl.BlockSpec((24, H), lambda i: (0, 0)),
            pl.BlockSpec((H, H), lambda i: (0, 0)),
            pl.BlockSpec((1, H), lambda i: (0, 0)),
            pl.BlockSpec((H, 8), lambda i: (0, 0)),
            pl.BlockSpec((1, 1), lambda i: (0, 0)),
            pl.BlockSpec((H, H), lambda i: (0, 0)),
            pl.BlockSpec((1, H), lambda i: (0, 0)),
            pl.BlockSpec((H, 8), lambda i: (0, 0)),
        ],
        out_specs=(pl.BlockSpec((EB, H), lambda i: (i, 0)),
                   pl.BlockSpec((EB, 16), lambda i: (i, 0))),
        out_shape=(jax.ShapeDtypeStruct((E, H), jnp.float32),
                   jax.ShapeDtypeStruct((E, 16), jnp.float32)),
    )(gdh, gsh, gx, edge_attr, wef, w2, b2, eiw, eib, xw1, xb1, xw2)


NP = 10240
HRANGE = NP // 2
TRASH = 128
RPS = HRANGE // NS
WB = 64
SEPW = E // NS
SNCHUNK = SEPW // CHUNK
NX = N * 4


def _scatter_body(msc_ref, mdxf_ref, dsti_ref, pm_ref, px_ref,
                  dstc, datm, datx1, idxm, zb, dxacc, accm):
    cid = lax.axis_index("c")
    sid = lax.axis_index("s")
    base = sid * SEPW
    lo = cid * HRANGE

    if True:
        def zrow(i, carry):
            for j in range(H // 16):
                zb[i, pl.ds(j * 16, 16)] = jnp.zeros((16,), jnp.float32)
            return carry

        lax.fori_loop(0, WB, zrow, 0)
        for r in range(RPS // WB):
            pltpu.sync_copy(zb, accm.at[pl.ds(sid * RPS + r * WB, WB)])

        @pl.when(cid == 0)
        def _():
            def zx(i, carry):
                dxacc[pl.ds(i * 16, 16)] = jnp.zeros((16,), jnp.float32)
                return carry

            lax.fori_loop(0, NX // 16, zx, 0)

        plsc.subcore_barrier()

        lane = lax.iota(jnp.int32, 16)
        lhi = lane >> 2
        llo = lane & 3

        def body(j, carry):
            pltpu.sync_copy(dsti_ref.at[pl.ds(base + j * CHUNK, CHUNK)], dstc)
            pltpu.sync_copy(msc_ref.at[pl.ds(base + j * CHUNK, CHUNK)], datm)
            for t in range(CHUNK // 16):
                iv = dstc[pl.ds(t * 16, 16)]
                ok = (iv >= lo) & (iv < lo + HRANGE)
                midx = jnp.where(ok, iv - lo, HRANGE + (iv & (TRASH - 1)))
                idxm[pl.ds(t * 16, 16)] = midx
            pltpu.sync_copy(datm, accm.at[idxm], add=True)

            @pl.when(cid == 0)
            def _():
                pltpu.sync_copy(
                    mdxf_ref.at[pl.ds((base + j * CHUNK) * 4, CHUNK * 4)],
                    datx1)
                for p in range(4):
                    for t in range(CHUNK // 16):
                        kv = p * (CHUNK // 4) + t * 4 + lhi
                        dv = plsc.load_gather(dstc, [kv])
                        val = datx1[pl.ds(p * CHUNK + t * 16, 16)]
                        plsc.addupdate_scatter(dxacc, [dv * 4 + llo], val)

            return carry

        lax.fori_loop(0, SNCHUNK, body, 0)
        plsc.subcore_barrier()

        for r in range(RPS // WB):
            rows = pl.ds(sid * RPS + r * WB, WB)
            pltpu.sync_copy(accm.at[rows], zb)
            pltpu.sync_copy(zb, pm_ref.at[cid, rows])

        @pl.when(cid == 0)
        def _():
            pltpu.sync_copy(dxacc, px_ref.at[sid])


def _sc_scatter(msc, mdxf, dsti):
    mesh = plsc.VectorSubcoreMesh(core_axis_name="c", subcore_axis_name="s",
                                  num_cores=NC, num_subcores=NS)
    return pl.kernel(
        _scatter_body,
        out_type=(jax.ShapeDtypeStruct((NC, HRANGE, H), jnp.float32),
                  jax.ShapeDtypeStruct((NS, NX), jnp.float32)),
        mesh=mesh,
        scratch_types=[
            pltpu.VMEM((CHUNK,), jnp.int32),
            pltpu.VMEM((CHUNK, H), jnp.float32),
            pltpu.VMEM((CHUNK * 4,), jnp.float32),
            pltpu.VMEM((CHUNK,), jnp.int32),
            pltpu.VMEM((WB, H), jnp.float32),
            pltpu.VMEM((NX,), jnp.float32),
            pltpu.VMEM_SHARED((HRANGE + TRASH, H), jnp.float32),
        ],
        compiler_params=pltpu.CompilerParams(needs_layout_passes=False),
    )(msc, mdxf, dsti)


def _node_body(pm_ref, h_ref, w1a_ref, w1b_ref, b1_ref, w2_ref,
               b2_ref, ho_ref):
    mi = pm_ref[...]
    h = h_ref[...]
    a = _silu(jnp.dot(mi, w1a_ref[...], preferred_element_type=jnp.float32)
              + jnp.dot(h, w1b_ref[...], preferred_element_type=jnp.float32)
              + b1_ref[...])
    ho_ref[...] = h + jnp.dot(a, w2_ref[...],
                              preferred_element_type=jnp.float32) + b2_ref[...]


def _node_mlp(pmr, h, w1a, w1b, b1, w2, b2):
    NB = 2000
    return pl.pallas_call(
        _node_body,
        grid=(N // NB,),
        in_specs=[
            pl.BlockSpec((NB, H), lambda i: (i, 0)),
            pl.BlockSpec((NB, H), lambda i: (i, 0)),
            pl.BlockSpec((H, H), lambda i: (0, 0)),
            pl.BlockSpec((H, H), lambda i: (0, 0)),
            pl.BlockSpec((1, H), lambda i: (0, 0)),
            pl.BlockSpec((H, H), lambda i: (0, 0)),
            pl.BlockSpec((1, H), lambda i: (0, 0)),
        ],
        out_specs=pl.BlockSpec((NB, H), lambda i: (i, 0)),
        out_shape=jax.ShapeDtypeStruct((N, H), jnp.float32),
    )(pmr, h, w1a, w1b, b1, w2, b2)


def _dx_sum_body(px_ref, dx_ref):
    dx_ref[...] = jnp.sum(px_ref[...], axis=0, keepdims=True)


def _dx_sum(px):
    return pl.pallas_call(
        _dx_sum_body,
        out_shape=jax.ShapeDtypeStruct((1, NX), jnp.float32),
    )(px)


def kernel(h, x, edge_index, edge_attr, em_w1, em_b1, em_w2, em_b2, ei_w,
           ei_b, xm_w1, xm_b1, xm_w2, nm_w1, nm_b1, nm_w2, nm_b2):
    x4 = jnp.pad(x, ((0, 0), (0, 1)))
    src = edge_index[0]
    dst = edge_index[1]

    w1a = em_w1[:H]
    w1b = em_w1[H:2 * H]
    wef = jnp.pad(em_w1[2 * H:], ((0, 7), (0, 0)))
    eiw = jnp.pad(ei_w, ((0, 0), (0, 7)))
    xw2 = jnp.pad(xm_w2, ((0, 0), (0, 7)))

    tdh, tsh = _make_tables(h, w1a, w1b, em_b1.reshape(1, H))
    gdh, gsh, gxf = _sc_gather(tdh, tsh, x4.reshape(-1), dst, src)
    gx = gxf.reshape(E, 4)
    msc, mdx = _edge_mlp(gdh, gsh, gx, edge_attr, wef, em_w2,
                         em_b2.reshape(1, H), eiw, ei_b.reshape(1, 1),
                         xm_w1, xm_b1.reshape(1, H), xw2)
    mdxf = mdx[:, :4].reshape(E * 4)
    pm, px = _sc_scatter(msc, mdxf, dst)
    ho = _node_mlp(pm.reshape(NP, H), h, nm_w1[:H], nm_w1[H:],
                   nm_b1.reshape(1, H), nm_w2, nm_b2.reshape(1, H))
    dx4 = _dx_sum(px)[0].reshape(N, 4)
    return (ho, x + dx4[:, :3])
